# baseline (speedup 1.0000x reference)
import functools

import jax
import jax.numpy as jnp
from jax import lax
from jax.experimental import pallas as pl
from jax.experimental.pallas import tpu as pltpu
from jax.experimental.pallas import tpu_sc as plsc

B = 4096
L = 50
D = 128
LAB = 128
N_ROWS = B * L

NUM_CORES = 2
NUM_SUBCORES = 16
NW = NUM_CORES * NUM_SUBCORES
BCHUNK = B // NW
LSTEP = 2
WROWS = LSTEP * BCHUNK
NSTEP = L // LSTEP
NBUF = 3


def _sc_gather_sum(emb_table, idx_pack):
    vector_mesh = plsc.VectorSubcoreMesh(
        core_axis_name="core", subcore_axis_name="subcore"
    )

    @functools.partial(
        pl.kernel,
        out_type=(
            jax.ShapeDtypeStruct((N_ROWS, D), jnp.float32),
            jax.ShapeDtypeStruct((B, D), jnp.float32),
        ),
        mesh=vector_mesh,
        scratch_types=(
            [pltpu.VMEM((WROWS,), jnp.int32) for _ in range(NBUF)]
            + [pltpu.VMEM((WROWS, D), jnp.float32) for _ in range(NBUF)]
            + [pltpu.VMEM((1, WROWS), jnp.int32)]
            + [pltpu.VMEM_SHARED((NUM_SUBCORES * BCHUNK, D), jnp.float32)]
            + [pltpu.SemaphoreType.DMA for _ in range(3 * NBUF + 1)]
        ),
    )
    def gather_kernel(table_hbm, idx_hbm, enc_hbm, sums_hbm, *scratch):
        idxb = scratch[0:NBUF]
        rows = scratch[NBUF : 2 * NBUF]
        accidx_v = scratch[2 * NBUF]
        shared_acc = scratch[2 * NBUF + 1]
        gsem = scratch[2 * NBUF + 2 : 2 * NBUF + 2 + NBUF]
        wsem = scratch[2 * NBUF + 2 + NBUF : 2 * NBUF + 2 + 2 * NBUF]
        asem = scratch[2 * NBUF + 2 + 2 * NBUF : 2 * NBUF + 2 + 3 * NBUF]
        isem = scratch[2 * NBUF + 2 + 3 * NBUF]

        sid = lax.axis_index("subcore")
        wid = sid * NUM_CORES + lax.axis_index("core")
        b_base = wid * BCHUNK

        for h in range(LSTEP):
            for c in range(BCHUNK // 16):
                accidx_v[0, pl.ds(h * BCHUNK + c * 16, 16)] = (
                    sid * BCHUNK + c * 16 + lax.iota(jnp.int32, 16)
                )

        zero = jnp.zeros((16,), jnp.float32)

        @pl.loop(0, BCHUNK)
        def _(r):
            for c in range(D // 16):
                rows[0][r, pl.ds(c * 16, 16)] = zero

        pltpu.sync_copy(
            rows[0].at[pl.ds(0, BCHUNK)],
            shared_acc.at[pl.ds(sid * BCHUNK, BCHUNK)],
        )

        def start_gather(s, j):
            pltpu.sync_copy(idx_hbm.at[wid, s], idxb[j])
            pltpu.make_async_copy(table_hbm.at[idxb[j]], rows[j], gsem[j]).start()

        def wait_gather(j):
            pltpu.make_async_copy(table_hbm.at[idxb[j]], rows[j], gsem[j]).wait()

        def wait_enc_writes(j):
            pltpu.make_async_copy(
                rows[j], enc_hbm.at[pl.ds(0, WROWS)], wsem[j]
            ).wait()

        def wait_acc(j):
            pltpu.make_async_copy(
                rows[j], shared_acc.at[pl.ds(0, WROWS)], asem[j]
            ).wait()

        def step(s, j):
            wait_gather(j)
            for h in range(LSTEP):
                pltpu.make_async_copy(
                    rows[j].at[pl.ds(h * BCHUNK, BCHUNK)],
                    enc_hbm.at[pl.ds((LSTEP * s + h) * B + b_base, BCHUNK)],
                    wsem[j],
                ).start()
            pltpu.async_copy(
                rows[j], shared_acc.at[accidx_v.at[0]], asem[j], add=True
            )

        def refill(s_next, j):
            wait_enc_writes(j)
            wait_acc(j)
            start_gather(s_next, j)

        for j in range(NBUF):
            start_gather(j, j)
        for s in range(NBUF):
            step(s, s)
            refill(s + NBUF, s)

        @pl.loop(NBUF, NSTEP - 1, step=NBUF)
        def _(s0):
            for j in range(NBUF):
                s = s0 + j
                step(s, j)

                @pl.when(s + NBUF < NSTEP)
                def _():
                    refill(s + NBUF, j)

        step(NSTEP - 1, (NSTEP - 1) % NBUF)

        for j in range(NBUF):
            wait_enc_writes(j)
            wait_acc(j)
        pltpu.sync_copy(
            shared_acc.at[pl.ds(sid * BCHUNK, BCHUNK)],
            sums_hbm.at[pl.ds(b_base, BCHUNK)],
        )

    return gather_kernel(emb_table, idx_pack)


def _tc_head_body(sums_ref, tgt_ref, w1t_ref, b1_ref, loss_ref):
    m = sums_ref[...] * (1.0 / L)
    r = jnp.tanh(
        jnp.dot(m, w1t_ref[...], preferred_element_type=jnp.float32)
        + b1_ref[...]
    )
    d = r - tgt_ref[...]
    loss_ref[...] = jnp.sum(d * d).reshape(1, 1)


def _tc_head(sums, tgt, W1t, b1):
    loss_sum = pl.pallas_call(
        _tc_head_body,
        out_shape=jax.ShapeDtypeStruct((1, 1), jnp.float32),
    )(sums, tgt, W1t, b1)
    return loss_sum[0, 0] / (B * LAB)


def kernel(x, tgt, emb_table, W1, b1):
    idx_lb = x.T.astype(jnp.int32)
    idx_pack = (
        idx_lb.reshape(NSTEP, LSTEP, NW, BCHUNK)
        .transpose(2, 0, 1, 3)
        .reshape(NW, NSTEP, WROWS)
    )
    enc_flat, sums = _sc_gather_sum(emb_table, idx_pack)
    loss = _tc_head(sums, tgt, W1.T, b1.reshape(1, LAB))
    enc_x = enc_flat.reshape(L, B, D).transpose(1, 0, 2)
    return (enc_x, loss)


# speedup vs baseline: 1.0552x; 1.0552x over previous
import functools

import jax
import jax.numpy as jnp
from jax import lax
from jax.experimental import pallas as pl
from jax.experimental.pallas import tpu as pltpu
from jax.experimental.pallas import tpu_sc as plsc

B = 4096
L = 50
D = 128
LAB = 128
N_ROWS = B * L

NUM_CORES = 2
NUM_SUBCORES = 16
NW = NUM_CORES * NUM_SUBCORES
BCHUNK = B // NW
NBUF = 6


def _sc_gather_sum(emb_table, idx_lb):
    vector_mesh = plsc.VectorSubcoreMesh(
        core_axis_name="core", subcore_axis_name="subcore"
    )

    @functools.partial(
        pl.kernel,
        out_type=(
            jax.ShapeDtypeStruct((N_ROWS, D), jnp.float32),
            jax.ShapeDtypeStruct((B, D), jnp.float32),
        ),
        mesh=vector_mesh,
        scratch_types=(
            [pltpu.VMEM((L, BCHUNK), jnp.int32)]
            + [pltpu.VMEM((BCHUNK, D), jnp.float32) for _ in range(NBUF)]
            + [pltpu.VMEM((1, BCHUNK), jnp.int32)]
            + [pltpu.VMEM_SHARED((NUM_SUBCORES * BCHUNK, D), jnp.float32)]
            + [pltpu.SemaphoreType.DMA for _ in range(3 * NBUF + 1)]
        ),
    )
    def gather_kernel(table_hbm, idx_hbm, enc_hbm, sums_hbm, *scratch):
        idx_v = scratch[0]
        rows = scratch[1 : 1 + NBUF]
        accidx_v = scratch[1 + NBUF]
        shared_acc = scratch[2 + NBUF]
        gsem = scratch[3 + NBUF : 3 + 2 * NBUF]
        wsem = scratch[3 + 2 * NBUF : 3 + 3 * NBUF]
        asem = scratch[3 + 3 * NBUF : 3 + 4 * NBUF]
        isem = scratch[3 + 4 * NBUF]

        sid = lax.axis_index("subcore")
        wid = sid * NUM_CORES + lax.axis_index("core")
        b_base = wid * BCHUNK

        for c in range(BCHUNK // 16):
            accidx_v[0, pl.ds(c * 16, 16)] = (
                sid * BCHUNK + c * 16 + lax.iota(jnp.int32, 16)
            )

        idx_dma = pltpu.async_copy(idx_hbm.at[:, pl.ds(b_base, BCHUNK)], idx_v, isem)

        zero = jnp.zeros((16,), jnp.float32)

        @pl.loop(0, BCHUNK)
        def _(r):
            for c in range(D // 16):
                rows[0][r, pl.ds(c * 16, 16)] = zero

        pltpu.sync_copy(
            rows[0], shared_acc.at[pl.ds(sid * BCHUNK, BCHUNK)]
        )
        idx_dma.wait()

        def start_gather(l, j):
            pltpu.make_async_copy(
                table_hbm.at[idx_v.at[l]], rows[j], gsem[j]
            ).start()

        def wait_gather(j):
            pltpu.make_async_copy(table_hbm.at[idx_v.at[0]], rows[j], gsem[j]).wait()

        def start_enc_write(l, j):
            pltpu.make_async_copy(
                rows[j], enc_hbm.at[pl.ds(l * B + b_base, BCHUNK)], wsem[j]
            ).start()

        def wait_enc_write(j):
            pltpu.make_async_copy(
                rows[j], enc_hbm.at[pl.ds(0, BCHUNK)], wsem[j]
            ).wait()

        def start_acc(j, add):
            pltpu.async_copy(
                rows[j], shared_acc.at[accidx_v.at[0]], asem[j], add=add
            )

        def wait_acc(j):
            pltpu.make_async_copy(
                rows[j], shared_acc.at[pl.ds(0, BCHUNK)], asem[j]
            ).wait()

        def step(l, j):
            wait_gather(j)
            start_enc_write(l, j)
            start_acc(j, add=True)

        def refill(l_next, j):
            wait_enc_write(j)
            wait_acc(j)
            start_gather(l_next, j)

        for j in range(NBUF):
            start_gather(j, j)
        for l in range(NBUF):
            step(l, l)
            refill(l + NBUF, l)

        @pl.loop(NBUF, L - 2, step=NBUF)
        def _(l0):
            for j in range(NBUF):
                l = l0 + j
                step(l, j)

                @pl.when(l + NBUF < L)
                def _():
                    refill(l + NBUF, j)

        step(L - 2, (L - 2) % NBUF)
        step(L - 1, (L - 1) % NBUF)

        for j in range(NBUF):
            wait_enc_write(j)
            wait_acc(j)
        pltpu.sync_copy(
            shared_acc.at[pl.ds(sid * BCHUNK, BCHUNK)],
            sums_hbm.at[pl.ds(b_base, BCHUNK)],
        )

    return gather_kernel(emb_table, idx_lb)


def _tc_head_body(sums_ref, tgt_ref, w1t_ref, b1_ref, loss_ref):
    m = sums_ref[...] * (1.0 / L)
    r = jnp.tanh(
        jnp.dot(m, w1t_ref[...], preferred_element_type=jnp.float32)
        + b1_ref[...]
    )
    d = r - tgt_ref[...]
    loss_ref[...] = jnp.sum(d * d).reshape(1, 1)


def _tc_head(sums, tgt, W1t, b1):
    loss_sum = pl.pallas_call(
        _tc_head_body,
        out_shape=jax.ShapeDtypeStruct((1, 1), jnp.float32),
    )(sums, tgt, W1t, b1)
    return loss_sum[0, 0] / (B * LAB)


def kernel(x, tgt, emb_table, W1, b1):
    idx_lb = x.T.astype(jnp.int32)
    enc_flat, sums = _sc_gather_sum(emb_table, idx_lb)
    loss = _tc_head(sums, tgt, W1.T, b1.reshape(1, LAB))
    enc_x = enc_flat.reshape(L, B, D).transpose(1, 0, 2)
    return (enc_x, loss)


# speedup vs baseline: 1.0769x; 1.0206x over previous
import functools

import jax
import jax.numpy as jnp
from jax import lax
from jax.experimental import pallas as pl
from jax.experimental.pallas import tpu as pltpu
from jax.experimental.pallas import tpu_sc as plsc

B = 4096
L = 50
D = 128
LAB = 128
N_ROWS = B * L

NUM_CORES = 2
NUM_SUBCORES = 16
NW = NUM_CORES * NUM_SUBCORES
BCHUNK = B // NW
NBUF = 5


def _sc_gather_sum(emb_table, idx_lb):
    vector_mesh = plsc.VectorSubcoreMesh(
        core_axis_name="core", subcore_axis_name="subcore"
    )

    @functools.partial(
        pl.kernel,
        out_type=(
            jax.ShapeDtypeStruct((N_ROWS, D), jnp.float32),
            jax.ShapeDtypeStruct((B, D), jnp.float32),
        ),
        mesh=vector_mesh,
        scratch_types=(
            [pltpu.VMEM((L, BCHUNK), jnp.int32)]
            + [pltpu.VMEM((BCHUNK, D), jnp.float32) for _ in range(NBUF)]
            + [pltpu.VMEM((1, BCHUNK), jnp.int32)]
            + [pltpu.VMEM_SHARED((NUM_SUBCORES * BCHUNK, D), jnp.float32)]
            + [pltpu.SemaphoreType.DMA for _ in range(3 * NBUF + 1)]
        ),
    )
    def gather_kernel(table_hbm, idx_hbm, enc_hbm, sums_hbm, *scratch):
        idx_v = scratch[0]
        rows = scratch[1 : 1 + NBUF]
        accidx_v = scratch[1 + NBUF]
        shared_acc = scratch[2 + NBUF]
        gsem = scratch[3 + NBUF : 3 + 2 * NBUF]
        wsem = scratch[3 + 2 * NBUF : 3 + 3 * NBUF]
        asem = scratch[3 + 3 * NBUF : 3 + 4 * NBUF]
        isem = scratch[3 + 4 * NBUF]

        sid = lax.axis_index("subcore")
        wid = sid * NUM_CORES + lax.axis_index("core")
        b_base = wid * BCHUNK

        for c in range(BCHUNK // 16):
            accidx_v[0, pl.ds(c * 16, 16)] = (
                sid * BCHUNK + c * 16 + lax.iota(jnp.int32, 16)
            )

        idx_dma = pltpu.async_copy(idx_hbm.at[:, pl.ds(b_base, BCHUNK)], idx_v, isem)

        zero = jnp.zeros((16,), jnp.float32)

        @pl.loop(0, BCHUNK)
        def _(r):
            for c in range(D // 16):
                rows[0][r, pl.ds(c * 16, 16)] = zero

        pltpu.sync_copy(rows[0], shared_acc.at[pl.ds(sid * BCHUNK, BCHUNK)])
        idx_dma.wait()

        def start_gather(l, j):
            pltpu.make_async_copy(
                table_hbm.at[idx_v.at[l]], rows[j], gsem[j]
            ).start()

        def wait_gather(j):
            pltpu.make_async_copy(table_hbm.at[idx_v.at[0]], rows[j], gsem[j]).wait()

        def start_enc_write(l, j):
            pltpu.make_async_copy(
                rows[j], enc_hbm.at[pl.ds(l * B + b_base, BCHUNK)], wsem[j]
            ).start()

        def wait_enc_write(j):
            pltpu.make_async_copy(
                rows[j], enc_hbm.at[pl.ds(0, BCHUNK)], wsem[j]
            ).wait()

        def start_acc(j, add):
            pltpu.async_copy(
                rows[j], shared_acc.at[accidx_v.at[0]], asem[j], add=add
            )

        def wait_acc(j):
            pltpu.make_async_copy(
                rows[j], shared_acc.at[pl.ds(0, BCHUNK)], asem[j]
            ).wait()

        for j in range(NBUF):
            start_gather(j, j)

        @pl.loop(0, L, step=NBUF)
        def _(l0):
            for j in range(NBUF):
                l = l0 + j
                wait_gather(j)
                start_enc_write(l, j)
                start_acc(j, add=True)

                @pl.when(l + NBUF < L)
                def _():
                    wait_enc_write(j)
                    wait_acc(j)
                    start_gather(l + NBUF, j)

        for j in range(NBUF):
            wait_enc_write(j)
            wait_acc(j)
        pltpu.sync_copy(
            shared_acc.at[pl.ds(sid * BCHUNK, BCHUNK)],
            sums_hbm.at[pl.ds(b_base, BCHUNK)],
        )

    return gather_kernel(emb_table, idx_lb)


def _tc_head_body(sums_ref, tgt_ref, w1t_ref, b1_ref, loss_ref):
    m = sums_ref[...] * (1.0 / L)
    r = jnp.tanh(
        jnp.dot(m, w1t_ref[...], preferred_element_type=jnp.float32)
        + b1_ref[...]
    )
    d = r - tgt_ref[...]
    loss_ref[...] = jnp.sum(d * d).reshape(1, 1)


def _tc_head(sums, tgt, W1t, b1):
    loss_sum = pl.pallas_call(
        _tc_head_body,
        out_shape=jax.ShapeDtypeStruct((1, 1), jnp.float32),
    )(sums, tgt, W1t, b1)
    return loss_sum[0, 0] / (B * LAB)


def kernel(x, tgt, emb_table, W1, b1):
    idx_lb = x.T.astype(jnp.int32)
    enc_flat, sums = _sc_gather_sum(emb_table, idx_lb)
    loss = _tc_head(sums, tgt, W1.T, b1.reshape(1, LAB))
    enc_x = enc_flat.reshape(L, B, D).transpose(1, 0, 2)
    return (enc_x, loss)
